# baseline (speedup 1.0000x reference)
import functools

import jax
import jax.numpy as jnp
from jax import lax
from jax.experimental import pallas as pl
from jax.experimental.pallas import tpu as pltpu
from jax.experimental.pallas import tpu_sc as plsc

N = 320000
NUM_SEG = 10000
D = 128
ZMAX = 100

NUM_TILES = 16
LANES = 128
ROWS = 2528
N_PAD = ROWS * LANES
R = 32
NB = ROWS // R
K_WIN = ROWS // NUM_TILES
N_ROWS_REAL = N // LANES


B = R * LANES


def _atom_scalar_body(z_ref, p3_ref, emb_ref, wp_ref,
                      wg_ref, w2_ref, y_ref):
    embT_h = (emb_ref[...].T * 0.5).astype(jnp.bfloat16)
    wpT_h = wp_ref[...].T * 0.5
    vT = (wg_ref[...] * w2_ref[...]).T
    types = lax.broadcasted_iota(jnp.int32, (ZMAX, LANES), 0)

    oh = jnp.concatenate(
        [(types == z_ref[g:g + 1, :]) for g in range(R)],
        axis=1).astype(jnp.bfloat16)
    pose = jnp.concatenate(
        [jnp.concatenate([p3_ref[k, g:g + 1, :] for g in range(R)], axis=1)
         for k in range(3)], axis=0)

    h = lax.dot_general(wpT_h, pose, (((1,), (0,)), ((), ())),
                        preferred_element_type=jnp.float32
                        ).astype(jnp.bfloat16)
    t = lax.tanh(h)
    lift_h = h * (t * jnp.bfloat16(0.5) + jnp.bfloat16(0.5))
    eg = lax.dot_general(embT_h, oh, (((1,), (0,)), ((), ())),
                         preferred_element_type=jnp.float32
                         ).astype(jnp.bfloat16)
    h2 = eg + lift_h
    su = h2 * lax.tanh(h2) + h2
    y = lax.dot_general(vT.astype(jnp.bfloat16), su,
                        (((1,), (0,)), ((), ())),
                        preferred_element_type=jnp.float32)
    y32 = jnp.concatenate(
        [y[:, g * LANES:(g + 1) * LANES] for g in range(R)], axis=0)
    grow = pl.program_id(0) * R + lax.broadcasted_iota(jnp.int32, (R, 1), 0)
    y_ref[...] = jnp.where(grow < N_ROWS_REAL, y32, 0.0)


def _atom_scalars(z2, p3, emb, wp, wg, w2):
    return pl.pallas_call(
        _atom_scalar_body,
        grid=(NB,),
        in_specs=[
            pl.BlockSpec((R, LANES), lambda i: (i, 0)),
            pl.BlockSpec((3, R, LANES), lambda i: (0, i, 0)),
            pl.BlockSpec((ZMAX, D), lambda i: (0, 0)),
            pl.BlockSpec((3, D), lambda i: (0, 0)),
            pl.BlockSpec((D, 1), lambda i: (0, 0)),
            pl.BlockSpec((D, 1), lambda i: (0, 0)),
        ],
        out_specs=pl.BlockSpec((R, LANES), lambda i: (i, 0)),
        out_shape=jax.ShapeDtypeStruct((ROWS, LANES), jnp.float32),
    )(z2, p3, emb, wp, wg, w2)


def _segsum_body(y_hbm, idx_hbm, init_hbm, out_hbm, yv, iv, acc):
    s = lax.axis_index("s")

    pltpu.sync_copy(y_hbm.at[s], yv)
    pltpu.sync_copy(idx_hbm.at[s], iv)

    @pl.when(s == 0)
    def _():
        pltpu.sync_copy(init_hbm, acc)

    plsc.subcore_barrier()

    def body(j, carry):
        pltpu.sync_copy(yv.at[j], acc.at[iv.at[j]], add=True)
        return carry

    lax.fori_loop(0, K_WIN, body, 0)

    plsc.subcore_barrier()

    @pl.when(s == 0)
    def _():
        pltpu.sync_copy(acc, out_hbm)


@functools.cache
def _build_segsum():
    return pl.kernel(
        _segsum_body,
        out_type=jax.ShapeDtypeStruct((NUM_SEG,), jnp.float32),
        mesh=plsc.VectorSubcoreMesh(core_axis_name="c", subcore_axis_name="s",
                                    num_cores=1, num_subcores=NUM_TILES),
        scratch_types=[
            pltpu.VMEM((K_WIN, LANES), jnp.float32),
            pltpu.VMEM((K_WIN, LANES), jnp.int32),
            pltpu.VMEM_SHARED((NUM_SEG,), jnp.float32),
        ],
    )


def kernel(z, pos, batch, embedding, Wp, w_gate, W2, b2):
    pad = N_PAD - N
    z2 = jnp.pad(z.astype(jnp.int32), (0, pad)).reshape(ROWS, LANES)
    p3 = jnp.pad(pos.T, ((0, 0), (0, pad))).reshape(3, ROWS, LANES)
    wg = w_gate.reshape(D, 1)

    y = _atom_scalars(z2, p3, embedding, Wp, wg, W2)

    y3 = y.reshape(NUM_TILES, LANES, K_WIN).swapaxes(1, 2)
    idx3 = jnp.pad(batch.astype(jnp.int32), (0, pad)).reshape(
        NUM_TILES, LANES, K_WIN).swapaxes(1, 2)
    init = jnp.broadcast_to(b2, (NUM_SEG,)).astype(jnp.float32)

    out = _build_segsum()(y3, idx3, init)
    return out.reshape(NUM_SEG, 1)


# speedup vs baseline: 1.0379x; 1.0379x over previous
import functools

import jax
import jax.numpy as jnp
from jax import lax
from jax.experimental import pallas as pl
from jax.experimental.pallas import tpu as pltpu
from jax.experimental.pallas import tpu_sc as plsc

N = 320000
NUM_SEG = 10000
D = 128
ZMAX = 100

NUM_TILES = 16
LANES = 128
ROWS = 2528
N_PAD = ROWS * LANES
R = 32
NB = ROWS // R
K_WIN = ROWS // NUM_TILES
N_ROWS_REAL = N // LANES


B = R * LANES


def _atom_scalar_body(z_ref, p3_ref, emb_ref, wp_ref,
                      wg_ref, w2_ref, y_ref):
    embT_h = (emb_ref[...].T * 0.5).astype(jnp.bfloat16)
    wpT_h = wp_ref[...].T * 0.5
    vT = (wg_ref[...] * w2_ref[...]).T
    types = lax.broadcasted_iota(jnp.int32, (ZMAX, LANES), 0)

    oh = jnp.concatenate(
        [(types == z_ref[g:g + 1, :]) for g in range(R)],
        axis=1).astype(jnp.bfloat16)
    pose = jnp.concatenate(
        [jnp.concatenate([p3_ref[k, g:g + 1, :] for g in range(R)], axis=1)
         for k in range(3)], axis=0)

    h = lax.dot_general(wpT_h, pose, (((1,), (0,)), ((), ())),
                        preferred_element_type=jnp.float32
                        ).astype(jnp.bfloat16)
    t = lax.tanh(h)
    lift_h = h * (t * jnp.bfloat16(0.5) + jnp.bfloat16(0.5))
    eg = lax.dot_general(embT_h, oh, (((1,), (0,)), ((), ())),
                         preferred_element_type=jnp.float32
                         ).astype(jnp.bfloat16)
    h2 = eg + lift_h
    su = h2 * lax.tanh(h2) + h2
    y = lax.dot_general(vT.astype(jnp.bfloat16), su,
                        (((1,), (0,)), ((), ())),
                        preferred_element_type=jnp.float32)
    y32 = jnp.concatenate(
        [y[:, g * LANES:(g + 1) * LANES] for g in range(R)], axis=0)
    grow = pl.program_id(0) * R + lax.broadcasted_iota(jnp.int32, (R, 1), 0)
    y_ref[...] = jnp.where(grow < N_ROWS_REAL, y32, 0.0)


def _atom_scalars(z2, p3, emb, wp, wg, w2):
    return pl.pallas_call(
        _atom_scalar_body,
        grid=(NB,),
        in_specs=[
            pl.BlockSpec((R, LANES), lambda i: (i, 0)),
            pl.BlockSpec((3, R, LANES), lambda i: (0, i, 0)),
            pl.BlockSpec((ZMAX, D), lambda i: (0, 0)),
            pl.BlockSpec((3, D), lambda i: (0, 0)),
            pl.BlockSpec((D, 1), lambda i: (0, 0)),
            pl.BlockSpec((D, 1), lambda i: (0, 0)),
        ],
        out_specs=pl.BlockSpec((R, LANES), lambda i: (i, 0)),
        out_shape=jax.ShapeDtypeStruct((ROWS, LANES), jnp.float32),
    )(z2, p3, emb, wp, wg, w2)


def _segsum_body(y_hbm, idx_hbm, init_hbm, out_hbm, yv, iv, acc):
    s = lax.axis_index("s")

    pltpu.sync_copy(y_hbm.at[s], yv)
    pltpu.sync_copy(idx_hbm.at[s], iv)

    @pl.when(s == 0)
    def _():
        pltpu.sync_copy(init_hbm, acc)

    plsc.subcore_barrier()

    def body(j, carry):
        pltpu.sync_copy(yv.at[j], acc.at[iv.at[j]], add=True)
        return carry

    lax.fori_loop(0, K_WIN, body, 0)

    plsc.subcore_barrier()

    @pl.when(s == 0)
    def _():
        pltpu.sync_copy(acc, out_hbm)


@functools.cache
def _build_segsum():
    return pl.kernel(
        _segsum_body,
        out_type=jax.ShapeDtypeStruct((NUM_SEG,), jnp.float32),
        mesh=plsc.VectorSubcoreMesh(core_axis_name="c", subcore_axis_name="s",
                                    num_cores=1, num_subcores=NUM_TILES),
        scratch_types=[
            pltpu.VMEM((K_WIN, LANES), jnp.float32),
            pltpu.VMEM((K_WIN, LANES), jnp.int32),
            pltpu.VMEM_SHARED((NUM_SEG,), jnp.float32),
        ],
    )


def kernel(z, pos, batch, embedding, Wp, w_gate, W2, b2):
    pad = N_PAD - N
    z2 = jnp.pad(z.astype(jnp.int32), (0, pad)).reshape(ROWS, LANES)
    p3 = jnp.pad(pos.T, ((0, 0), (0, pad))).reshape(3, ROWS, LANES)
    wg = w_gate.reshape(D, 1)

    y = _atom_scalars(z2, p3, embedding, Wp, wg, W2)

    y3 = y.reshape(NUM_TILES, K_WIN, LANES)
    idx3 = jnp.pad(batch.astype(jnp.int32), (0, pad)).reshape(
        NUM_TILES, K_WIN, LANES)
    init = jnp.broadcast_to(b2, (NUM_SEG,)).astype(jnp.float32)

    out = _build_segsum()(y3, idx3, init)
    return out.reshape(NUM_SEG, 1)


# speedup vs baseline: 1.0562x; 1.0176x over previous
import functools

import jax
import jax.numpy as jnp
from jax import lax
from jax.experimental import pallas as pl
from jax.experimental.pallas import tpu as pltpu
from jax.experimental.pallas import tpu_sc as plsc

N = 320000
NUM_SEG = 10000
D = 128
ZMAX = 100

NUM_TILES = 16
LANES = 128
ROWS = 2528
N_PAD = ROWS * LANES
R = 32
NB = ROWS // R
K_WIN = ROWS // NUM_TILES
N_ROWS_REAL = N // LANES


B = R * LANES


def _atom_scalar_body(z_ref, p3_ref, emb_ref, wp_ref,
                      wg_ref, w2_ref, y_ref):
    embT_h = (emb_ref[...].T * 0.5).astype(jnp.bfloat16)
    wpT_h = wp_ref[...].T * 0.5
    vT = (wg_ref[...] * w2_ref[...]).T
    types = lax.broadcasted_iota(jnp.int32, (ZMAX, LANES), 0)

    oh = jnp.concatenate(
        [(types == z_ref[g:g + 1, :]) for g in range(R)],
        axis=1).astype(jnp.bfloat16)
    pose = jnp.concatenate(
        [jnp.concatenate([p3_ref[k, g:g + 1, :] for g in range(R)], axis=1)
         for k in range(3)], axis=0)

    h = lax.dot_general(wpT_h, pose, (((1,), (0,)), ((), ())),
                        preferred_element_type=jnp.float32
                        ).astype(jnp.bfloat16)
    t = lax.tanh(h)
    lift_h = h * (t * jnp.bfloat16(0.5) + jnp.bfloat16(0.5))
    eg = lax.dot_general(embT_h, oh, (((1,), (0,)), ((), ())),
                         preferred_element_type=jnp.float32
                         ).astype(jnp.bfloat16)
    h2 = eg + lift_h
    su = h2 * lax.tanh(h2) + h2
    y = lax.dot_general(vT.astype(jnp.bfloat16), su,
                        (((1,), (0,)), ((), ())),
                        preferred_element_type=jnp.float32)
    y32 = jnp.concatenate(
        [y[:, g * LANES:(g + 1) * LANES] for g in range(R)], axis=0)
    grow = pl.program_id(0) * R + lax.broadcasted_iota(jnp.int32, (R, 1), 0)
    y_ref[...] = jnp.where(grow < N_ROWS_REAL, y32, 0.0)


def _atom_scalars(z2, p3, emb, wp, wg, w2):
    return pl.pallas_call(
        _atom_scalar_body,
        grid=(NB,),
        in_specs=[
            pl.BlockSpec((R, LANES), lambda i: (i, 0)),
            pl.BlockSpec((3, R, LANES), lambda i: (0, i, 0)),
            pl.BlockSpec((ZMAX, D), lambda i: (0, 0)),
            pl.BlockSpec((3, D), lambda i: (0, 0)),
            pl.BlockSpec((D, 1), lambda i: (0, 0)),
            pl.BlockSpec((D, 1), lambda i: (0, 0)),
        ],
        out_specs=pl.BlockSpec((R, LANES), lambda i: (i, 0)),
        out_shape=jax.ShapeDtypeStruct((ROWS, LANES), jnp.float32),
    )(z2, p3, emb, wp, wg, w2)


def _segsum_body(y_hbm, idx_hbm, init_hbm, out_hbm, yv, iv, acc):
    s = lax.axis_index("s")

    pltpu.sync_copy(y_hbm.at[s], yv)
    pltpu.sync_copy(idx_hbm.at[s], iv)

    @pl.when(s == 0)
    def _():
        pltpu.sync_copy(init_hbm, acc)

    plsc.subcore_barrier()

    pltpu.sync_copy(yv, acc.at[iv], add=True)

    plsc.subcore_barrier()

    @pl.when(s == 0)
    def _():
        pltpu.sync_copy(acc, out_hbm)


@functools.cache
def _build_segsum():
    return pl.kernel(
        _segsum_body,
        out_type=jax.ShapeDtypeStruct((NUM_SEG,), jnp.float32),
        mesh=plsc.VectorSubcoreMesh(core_axis_name="c", subcore_axis_name="s",
                                    num_cores=1, num_subcores=NUM_TILES),
        scratch_types=[
            pltpu.VMEM((K_WIN * LANES,), jnp.float32),
            pltpu.VMEM((K_WIN * LANES,), jnp.int32),
            pltpu.VMEM_SHARED((NUM_SEG,), jnp.float32),
        ],
    )


def kernel(z, pos, batch, embedding, Wp, w_gate, W2, b2):
    pad = N_PAD - N
    z2 = jnp.pad(z.astype(jnp.int32), (0, pad)).reshape(ROWS, LANES)
    p3 = jnp.pad(pos.T, ((0, 0), (0, pad))).reshape(3, ROWS, LANES)
    wg = w_gate.reshape(D, 1)

    y = _atom_scalars(z2, p3, embedding, Wp, wg, W2)

    y3 = y.reshape(NUM_TILES, K_WIN * LANES)
    idx3 = jnp.pad(batch.astype(jnp.int32), (0, pad)).reshape(
        NUM_TILES, K_WIN * LANES)
    init = jnp.broadcast_to(b2, (NUM_SEG,)).astype(jnp.float32)

    out = _build_segsum()(y3, idx3, init)
    return out.reshape(NUM_SEG, 1)
